# baseline (speedup 1.0000x reference)
import jax
import jax.numpy as jnp
from jax import lax
from jax.experimental import pallas as pl
from jax.experimental.pallas import tpu as pltpu
from jax.experimental.pallas import tpu_sc as plsc

NC = 2
NS = 16

N_USERS = 16384
N_GROUPS = 1024
EMB = 64
HEMB = EMB // NC
N_BEH = 819200

B = 128
BEH_PER_W = N_BEH // NS
NBATCH = BEH_PER_W // B
SB = 8
NSB = NBATCH // SB
UPT = N_USERS // NS
UB = 128
NUB = UPT // UB
GPT = N_GROUPS // NS
KV = HEMB // 16


def _lane_bcast(v, l):
  idx = jnp.full((16, 1), l, dtype=jnp.int32)
  dn = lax.GatherDimensionNumbers(
      offset_dims=(), collapsed_slice_dims=(0,), start_index_map=(0,))
  return lax.gather(v, idx, dn, slice_sizes=(1,),
                    mode=lax.GatherScatterMode.PROMISE_IN_BOUNDS)


def _sc_body(bidx_h, bcnt_h, bseg_h, uidx_h, gidx_h, ut_h, it_h, out_h,
             bidx_s, bcnt_s, bseg_s, rows, half, uidx, gidx,
             ubeh, gacc, gsem):
  cid = lax.axis_index("c")
  sid = lax.axis_index("s")
  col0 = cid * HEMB

  zero16 = jnp.zeros((16,), jnp.float32)

  def zrow(r, _):
    for k in range(KV):
      half[r, pl.ds(k * 16, 16)] = zero16
    return 0
  lax.fori_loop(0, GPT, zrow, 0)

  def zcopy(j, _):
    pltpu.sync_copy(half.at[pl.ds(0, GPT)],
                    ubeh.at[pl.ds(sid * UPT + j * GPT, GPT)])
    return 0
  lax.fori_loop(0, UPT // GPT, zcopy, 0)
  pltpu.sync_copy(half.at[pl.ds(0, GPT)], gacc.at[pl.ds(sid * GPT, GPT)])

  plsc.subcore_barrier()

  def superstep(s, _):
    pltpu.sync_copy(bidx_h.at[sid, pl.ds(s * SB, SB)], bidx_s)
    pltpu.sync_copy(bcnt_h.at[sid, pl.ds(s * SB, SB)], bcnt_s)
    pltpu.sync_copy(bseg_h.at[sid, pl.ds(s * SB, SB)], bseg_s)

    pltpu.async_copy(it_h.at[bidx_s.at[0]], rows.at[0], gsem)

    def pairstep(jj, _):
      for p in range(2):
        j = jj * 2 + p
        pltpu.make_async_copy(it_h.at[bidx_s.at[j]], rows.at[p], gsem).wait()

        @pl.when(j + 1 < SB)
        def _prefetch():
          pltpu.async_copy(it_h.at[bidx_s.at[j + 1]], rows.at[1 - p], gsem)

        def blk16(i, _, p=p, j=j):
          base = pl.multiple_of(i * 16, 16)
          cvec = bcnt_s[j, pl.ds(base, 16)]
          ivec = bidx_s[j, pl.ds(base, 16)]
          cvec = jnp.where(ivec != 0, cvec, 0.0)
          for l in range(16):
            cl = _lane_bcast(cvec, l)
            r = i * 16 + l
            for k in range(KV):
              src = pl.multiple_of(col0 + k * 16, 16)
              half[r, pl.ds(k * 16, 16)] = rows[p, r, pl.ds(src, 16)] * cl
          return 0
        lax.fori_loop(0, B // 16, blk16, 0)
        pltpu.sync_copy(half, ubeh.at[bseg_s.at[j]], add=True)
      return 0
    lax.fori_loop(0, SB // 2, pairstep, 0)
    return 0
  lax.fori_loop(0, NSB, superstep, 0)

  plsc.subcore_barrier()

  pltpu.sync_copy(uidx_h.at[sid], uidx)
  pltpu.sync_copy(gidx_h.at[sid], gidx)

  def ubatch(j, _):
    pltpu.async_copy(ut_h.at[uidx.at[j]], rows.at[0], gsem).wait()
    pltpu.sync_copy(ubeh.at[pl.ds(sid * UPT + j * UB, UB)], half)

    def blk16(i, _):
      base = pl.multiple_of(i * 16, 16)
      uvec = uidx[j, pl.ds(base, 16)]
      mvec = jnp.where(uvec != 0,
                       jnp.ones((16,), jnp.float32),
                       jnp.zeros((16,), jnp.float32))
      for l in range(16):
        ml = _lane_bcast(mvec, l)
        r = i * 16 + l
        for k in range(KV):
          src = pl.multiple_of(col0 + k * 16, 16)
          half[r, pl.ds(k * 16, 16)] = (
              half[r, pl.ds(k * 16, 16)] * rows[0, r, pl.ds(src, 16)] * ml)
      return 0
    lax.fori_loop(0, UB // 16, blk16, 0)
    pltpu.sync_copy(half, gacc.at[gidx.at[j]], add=True)
    return 0
  lax.fori_loop(0, NUB, ubatch, 0)

  plsc.subcore_barrier()

  pltpu.sync_copy(gacc.at[pl.ds(sid * GPT, GPT)], half.at[pl.ds(0, GPT)])
  pltpu.sync_copy(half.at[pl.ds(0, GPT)], out_h.at[cid, pl.ds(sid * GPT, GPT)])


def kernel(user_ids, group_segment_ids, item_ids, counts, user_segment_ids,
           user_table, item_table):
  bidx = item_ids.astype(jnp.int32).reshape(NS, NBATCH, B)
  bcnt = counts.reshape(NS, NBATCH, B)
  bseg = user_segment_ids.astype(jnp.int32).reshape(NS, NBATCH, B)
  uidx = user_ids.astype(jnp.int32).reshape(NS, NUB, UB)
  gidx = group_segment_ids.astype(jnp.int32).reshape(NS, NUB, UB)

  mesh = plsc.VectorSubcoreMesh(
      core_axis_name="c", subcore_axis_name="s", num_cores=NC, num_subcores=NS)
  sc = pl.kernel(
      _sc_body,
      out_type=jax.ShapeDtypeStruct((NC, N_GROUPS, HEMB), jnp.float32),
      mesh=mesh,
      scratch_types=[
          pltpu.VMEM((SB, B), jnp.int32),
          pltpu.VMEM((SB, B), jnp.float32),
          pltpu.VMEM((SB, B), jnp.int32),
          pltpu.VMEM((2, B, EMB), jnp.float32),
          pltpu.VMEM((B, HEMB), jnp.float32),
          pltpu.VMEM((NUB, UB), jnp.int32),
          pltpu.VMEM((NUB, UB), jnp.int32),
          pltpu.VMEM_SHARED((N_USERS, HEMB), jnp.float32),
          pltpu.VMEM_SHARED((N_GROUPS, HEMB), jnp.float32),
          pltpu.SemaphoreType.DMA,
      ],
      compiler_params=pltpu.CompilerParams(use_tc_tiling_on_sc=False),
  )
  halves = sc(bidx, bcnt, bseg, uidx, gidx, user_table, item_table)
  return jnp.concatenate([halves[0], halves[1]], axis=1)


# speedup vs baseline: 1.2237x; 1.2237x over previous
import jax
import jax.numpy as jnp
from jax import lax
from jax.experimental import pallas as pl
from jax.experimental.pallas import tpu as pltpu
from jax.experimental.pallas import tpu_sc as plsc

NC = 2
NS = 16

N_USERS = 16384
N_GROUPS = 1024
EMB = 64
HEMB = EMB // NC
N_BEH = 819200

B = 128
BEH_PER_W = N_BEH // NS
NBATCH = BEH_PER_W // B
SB = 20
NSB = NBATCH // SB
UPT = N_USERS // NS
UB = 128
NUB = UPT // UB
GPT = N_GROUPS // NS
KV = HEMB // 16


def _lane_bcast(v, l):
  idx = jnp.full((16, 1), l, dtype=jnp.int32)
  dn = lax.GatherDimensionNumbers(
      offset_dims=(), collapsed_slice_dims=(0,), start_index_map=(0,))
  return lax.gather(v, idx, dn, slice_sizes=(1,),
                    mode=lax.GatherScatterMode.PROMISE_IN_BOUNDS)


def _sc_body(bidx_h, bcnt_h, bseg_h, uidx_h, gidx_h, ut_h, it_h, out_h,
             bidx_s, bcnt_s, bseg_s, rows, uidx, gidx, umsk, ubuf, urows,
             ubeh, gacc, gsem):
  cid = lax.axis_index("c")
  sid = lax.axis_index("s")

  zero16 = jnp.zeros((16,), jnp.float32)

  def zrow(r, _):
    for k in range(KV):
      urows[r, pl.ds(k * 16, 16)] = zero16
    return 0
  lax.fori_loop(0, GPT, zrow, 0)

  def zcopy(j, _):
    pltpu.sync_copy(urows.at[pl.ds(0, GPT)],
                    ubeh.at[pl.ds(sid * UPT + j * GPT, GPT)])
    return 0
  lax.fori_loop(0, UPT // GPT, zcopy, 0)
  pltpu.sync_copy(urows.at[pl.ds(0, GPT)], gacc.at[pl.ds(sid * GPT, GPT)])

  plsc.subcore_barrier()

  def superstep(s, _):
    pltpu.sync_copy(bidx_h.at[sid, pl.ds(s * SB, SB)], bidx_s)
    pltpu.sync_copy(bcnt_h.at[sid, pl.ds(s * SB, SB)], bcnt_s)
    pltpu.sync_copy(bseg_h.at[sid, pl.ds(s * SB, SB)], bseg_s)

    def btrans(i, _):
      r = i // (B // 16)
      c0 = pl.multiple_of((i % (B // 16)) * 16, 16)
      iv = bidx_s[r, pl.ds(c0, 16)]
      cv = bcnt_s[r, pl.ds(c0, 16)]
      bcnt_s[r, pl.ds(c0, 16)] = jnp.where(iv != 0, cv, 0.0)
      bidx_s[r, pl.ds(c0, 16)] = iv * NC + cid
      return 0
    lax.fori_loop(0, SB * B // 16, btrans, 0)

    pltpu.async_copy(it_h.at[bidx_s.at[0]], rows.at[0], gsem)

    def pairstep(jj, _):
      for p in range(2):
        j = jj * 2 + p
        pltpu.make_async_copy(it_h.at[bidx_s.at[j]], rows.at[p], gsem).wait()

        @pl.when(j + 1 < SB)
        def _prefetch():
          pltpu.async_copy(it_h.at[bidx_s.at[j + 1]], rows.at[1 - p], gsem)

        def blk16(i, _, p=p, j=j):
          base = pl.multiple_of(i * 16, 16)
          cvec = bcnt_s[j, pl.ds(base, 16)]
          for l in range(16):
            cl = _lane_bcast(cvec, l)
            r = i * 16 + l
            for k in range(KV):
              rows[p, r, pl.ds(k * 16, 16)] = rows[p, r, pl.ds(k * 16, 16)] * cl
          return 0
        lax.fori_loop(0, B // 16, blk16, 0)
        pltpu.sync_copy(rows.at[p], ubeh.at[bseg_s.at[j]], add=True)
      return 0
    lax.fori_loop(0, SB // 2, pairstep, 0)
    return 0
  lax.fori_loop(0, NSB, superstep, 0)

  plsc.subcore_barrier()

  pltpu.sync_copy(uidx_h.at[sid], uidx)
  pltpu.sync_copy(gidx_h.at[sid], gidx)

  def utrans(i, _):
    r = i // (UB // 16)
    c0 = pl.multiple_of((i % (UB // 16)) * 16, 16)
    uv = uidx[r, pl.ds(c0, 16)]
    umsk[r, pl.ds(c0, 16)] = jnp.where(
        uv != 0, jnp.ones((16,), jnp.float32), jnp.zeros((16,), jnp.float32))
    uidx[r, pl.ds(c0, 16)] = uv * NC + cid
    return 0
  lax.fori_loop(0, UPT // 16, utrans, 0)

  def ubatch(j, _):
    pltpu.async_copy(ut_h.at[uidx.at[j]], urows, gsem).wait()
    pltpu.sync_copy(ubeh.at[pl.ds(sid * UPT + j * UB, UB)], ubuf)

    def blk16(i, _):
      base = pl.multiple_of(i * 16, 16)
      mvec = umsk[j, pl.ds(base, 16)]
      for l in range(16):
        ml = _lane_bcast(mvec, l)
        r = i * 16 + l
        for k in range(KV):
          urows[r, pl.ds(k * 16, 16)] = (
              urows[r, pl.ds(k * 16, 16)] * ubuf[r, pl.ds(k * 16, 16)] * ml)
      return 0
    lax.fori_loop(0, UB // 16, blk16, 0)
    pltpu.sync_copy(urows, gacc.at[gidx.at[j]], add=True)
    return 0
  lax.fori_loop(0, NUB, ubatch, 0)

  plsc.subcore_barrier()

  pltpu.sync_copy(gacc.at[pl.ds(sid * GPT, GPT)], urows.at[pl.ds(0, GPT)])
  pltpu.sync_copy(urows.at[pl.ds(0, GPT)], out_h.at[cid, pl.ds(sid * GPT, GPT)])


def kernel(user_ids, group_segment_ids, item_ids, counts, user_segment_ids,
           user_table, item_table):
  bidx = item_ids.astype(jnp.int32).reshape(NS, NBATCH, B)
  bcnt = counts.reshape(NS, NBATCH, B)
  bseg = user_segment_ids.astype(jnp.int32).reshape(NS, NBATCH, B)
  uidx = user_ids.astype(jnp.int32).reshape(NS, NUB, UB)
  gidx = group_segment_ids.astype(jnp.int32).reshape(NS, NUB, UB)
  ut2 = user_table.reshape(-1, HEMB)
  it2 = item_table.reshape(-1, HEMB)

  mesh = plsc.VectorSubcoreMesh(
      core_axis_name="c", subcore_axis_name="s", num_cores=NC, num_subcores=NS)
  sc = pl.kernel(
      _sc_body,
      out_type=jax.ShapeDtypeStruct((NC, N_GROUPS, HEMB), jnp.float32),
      mesh=mesh,
      scratch_types=[
          pltpu.VMEM((SB, B), jnp.int32),
          pltpu.VMEM((SB, B), jnp.float32),
          pltpu.VMEM((SB, B), jnp.int32),
          pltpu.VMEM((2, B, HEMB), jnp.float32),
          pltpu.VMEM((NUB, UB), jnp.int32),
          pltpu.VMEM((NUB, UB), jnp.int32),
          pltpu.VMEM((NUB, UB), jnp.float32),
          pltpu.VMEM((UB, HEMB), jnp.float32),
          pltpu.VMEM((UB, HEMB), jnp.float32),
          pltpu.VMEM_SHARED((N_USERS, HEMB), jnp.float32),
          pltpu.VMEM_SHARED((N_GROUPS, HEMB), jnp.float32),
          pltpu.SemaphoreType.DMA,
      ],
      compiler_params=pltpu.CompilerParams(use_tc_tiling_on_sc=False),
  )
  halves = sc(bidx, bcnt, bseg, uidx, gidx, ut2, it2)
  return jnp.concatenate([halves[0], halves[1]], axis=1)


# speedup vs baseline: 1.5089x; 1.2331x over previous
import jax
import jax.numpy as jnp
from jax import lax
from jax.experimental import pallas as pl
from jax.experimental.pallas import tpu as pltpu
from jax.experimental.pallas import tpu_sc as plsc

NC = 2
NS = 16

N_USERS = 16384
N_GROUPS = 1024
EMB = 64
HEMB = EMB // NC
N_BEH = 819200

B = 128
BEH_PER_W = N_BEH // NS
NBATCH = BEH_PER_W // B
SB = 20
NSB = NBATCH // SB
UPT = N_USERS // NS
UB = 128
NUB = UPT // UB
GPT = N_GROUPS // NS
KV = HEMB // 16


def _lane_bcast(v, l):
  idx = jnp.full((16, 1), l, dtype=jnp.int32)
  dn = lax.GatherDimensionNumbers(
      offset_dims=(), collapsed_slice_dims=(0,), start_index_map=(0,))
  return lax.gather(v, idx, dn, slice_sizes=(1,),
                    mode=lax.GatherScatterMode.PROMISE_IN_BOUNDS)


def _p1_body(bidx_h, bcnt_h, bseg_h, it_h, ub_out,
             bidx_s, bcnt_s, bseg_s, rows, urows, ubeh, gsem):
  cid = lax.axis_index("c")
  sid = lax.axis_index("s")

  zero16 = jnp.zeros((16,), jnp.float32)

  def zrow(r, _):
    for k in range(KV):
      urows[r, pl.ds(k * 16, 16)] = zero16
    return 0
  lax.fori_loop(0, GPT, zrow, 0)

  def zcopy(j, _):
    pltpu.sync_copy(urows.at[pl.ds(0, GPT)],
                    ubeh.at[pl.ds(sid * UPT + j * GPT, GPT)])
    return 0
  lax.fori_loop(0, UPT // GPT, zcopy, 0)

  plsc.subcore_barrier()

  def superstep(s, _):
    pltpu.sync_copy(bidx_h.at[sid, pl.ds(s * SB, SB)], bidx_s)
    pltpu.sync_copy(bcnt_h.at[sid, pl.ds(s * SB, SB)], bcnt_s)
    pltpu.sync_copy(bseg_h.at[sid, pl.ds(s * SB, SB)], bseg_s)

    def btrans(i, _):
      r = i // (B // 16)
      c0 = pl.multiple_of((i % (B // 16)) * 16, 16)
      iv = bidx_s[r, pl.ds(c0, 16)]
      cv = bcnt_s[r, pl.ds(c0, 16)]
      bcnt_s[r, pl.ds(c0, 16)] = jnp.where(iv != 0, cv, 0.0)
      bidx_s[r, pl.ds(c0, 16)] = iv * NC + cid
      return 0
    lax.fori_loop(0, SB * B // 16, btrans, 0)

    pltpu.async_copy(it_h.at[bidx_s.at[0]], rows.at[0], gsem)

    def pairstep(jj, _):
      for p in range(2):
        j = jj * 2 + p
        pltpu.make_async_copy(it_h.at[bidx_s.at[j]], rows.at[p], gsem).wait()

        @pl.when(j + 1 < SB)
        def _prefetch():
          pltpu.async_copy(it_h.at[bidx_s.at[j + 1]], rows.at[1 - p], gsem)

        def blk16(i, _, p=p, j=j):
          base = pl.multiple_of(i * 16, 16)
          cvec = bcnt_s[j, pl.ds(base, 16)]
          for l in range(16):
            cl = _lane_bcast(cvec, l)
            r = i * 16 + l
            for k in range(KV):
              rows[p, r, pl.ds(k * 16, 16)] = rows[p, r, pl.ds(k * 16, 16)] * cl
          return 0
        lax.fori_loop(0, B // 16, blk16, 0)
        pltpu.sync_copy(rows.at[p], ubeh.at[bseg_s.at[j]], add=True)
      return 0
    lax.fori_loop(0, SB // 2, pairstep, 0)
    return 0
  lax.fori_loop(0, NSB, superstep, 0)

  plsc.subcore_barrier()

  def udump(j, _):
    pltpu.sync_copy(ubeh.at[pl.ds(sid * UPT + j * UB, UB)],
                    urows.at[pl.ds(0, UB)])
    pltpu.sync_copy(urows.at[pl.ds(0, UB)],
                    ub_out.at[cid, pl.ds(sid * UPT + j * UB, UB)])
    return 0
  lax.fori_loop(0, UPT // UB, udump, 0)


def _p2_body(uidx_h, gidx_h, ut_h, ubeh_h, out_h,
             uidx, gidx, umsk, ubuf, urows, gacc, gsem):
  cid = lax.axis_index("c")
  sid = lax.axis_index("s")

  zero16 = jnp.zeros((16,), jnp.float32)

  def zrow(r, _):
    for k in range(KV):
      urows[r, pl.ds(k * 16, 16)] = zero16
    return 0
  lax.fori_loop(0, GPT, zrow, 0)
  pltpu.sync_copy(urows.at[pl.ds(0, GPT)], gacc.at[pl.ds(sid * GPT, GPT)])

  plsc.subcore_barrier()

  pltpu.sync_copy(uidx_h.at[sid], uidx)
  pltpu.sync_copy(gidx_h.at[sid], gidx)

  def utrans(i, _):
    r = i // (UB // 16)
    c0 = pl.multiple_of((i % (UB // 16)) * 16, 16)
    uv = uidx[r, pl.ds(c0, 16)]
    umsk[r, pl.ds(c0, 16)] = jnp.where(
        uv != 0, jnp.ones((16,), jnp.float32), jnp.zeros((16,), jnp.float32))
    uidx[r, pl.ds(c0, 16)] = uv * NC + cid
    return 0
  lax.fori_loop(0, UPT // 16, utrans, 0)

  def ubatch(j, _):
    pltpu.async_copy(ut_h.at[uidx.at[j]], urows, gsem).wait()
    pltpu.sync_copy(ubeh_h.at[cid, pl.ds(sid * UPT + j * UB, UB)], ubuf)

    def blk16(i, _):
      base = pl.multiple_of(i * 16, 16)
      mvec = umsk[j, pl.ds(base, 16)]
      for l in range(16):
        ml = _lane_bcast(mvec, l)
        r = i * 16 + l
        for k in range(KV):
          urows[r, pl.ds(k * 16, 16)] = (
              urows[r, pl.ds(k * 16, 16)] * ubuf[r, pl.ds(k * 16, 16)] * ml)
      return 0
    lax.fori_loop(0, UB // 16, blk16, 0)
    pltpu.sync_copy(urows, gacc.at[gidx.at[j]], add=True)
    return 0
  lax.fori_loop(0, NUB, ubatch, 0)

  plsc.subcore_barrier()

  pltpu.sync_copy(gacc.at[pl.ds(sid * GPT, GPT)], urows.at[pl.ds(0, GPT)])
  pltpu.sync_copy(urows.at[pl.ds(0, GPT)], out_h.at[cid, pl.ds(sid * GPT, GPT)])


def kernel(user_ids, group_segment_ids, item_ids, counts, user_segment_ids,
           user_table, item_table):
  bidx = item_ids.astype(jnp.int32).reshape(NS, NBATCH, B)
  bcnt = counts.reshape(NS, NBATCH, B)
  bseg = user_segment_ids.astype(jnp.int32).reshape(NS, NBATCH, B)
  uidx = user_ids.astype(jnp.int32).reshape(NS, NUB, UB)
  gidx = group_segment_ids.astype(jnp.int32).reshape(NS, NUB, UB)
  ut2 = user_table.reshape(-1, HEMB)
  it2 = item_table.reshape(-1, HEMB)

  mesh = plsc.VectorSubcoreMesh(
      core_axis_name="c", subcore_axis_name="s", num_cores=NC, num_subcores=NS)
  p1 = pl.kernel(
      _p1_body,
      out_type=jax.ShapeDtypeStruct((NC, N_USERS, HEMB), jnp.float32),
      mesh=mesh,
      scratch_types=[
          pltpu.VMEM((SB, B), jnp.int32),
          pltpu.VMEM((SB, B), jnp.float32),
          pltpu.VMEM((SB, B), jnp.int32),
          pltpu.VMEM((2, B, HEMB), jnp.float32),
          pltpu.VMEM((UB, HEMB), jnp.float32),
          pltpu.VMEM_SHARED((N_USERS, HEMB), jnp.float32),
          pltpu.SemaphoreType.DMA,
      ],
      compiler_params=pltpu.CompilerParams(use_tc_tiling_on_sc=False),
  )
  ubeh_hbm = p1(bidx, bcnt, bseg, it2)

  p2 = pl.kernel(
      _p2_body,
      out_type=jax.ShapeDtypeStruct((NC, N_GROUPS, HEMB), jnp.float32),
      mesh=mesh,
      scratch_types=[
          pltpu.VMEM((NUB, UB), jnp.int32),
          pltpu.VMEM((NUB, UB), jnp.int32),
          pltpu.VMEM((NUB, UB), jnp.float32),
          pltpu.VMEM((UB, HEMB), jnp.float32),
          pltpu.VMEM((UB, HEMB), jnp.float32),
          pltpu.VMEM_SHARED((N_GROUPS, HEMB), jnp.float32),
          pltpu.SemaphoreType.DMA,
      ],
      compiler_params=pltpu.CompilerParams(use_tc_tiling_on_sc=False),
  )
  halves = p2(uidx, gidx, ut2, ubeh_hbm)
  return jnp.concatenate([halves[0], halves[1]], axis=1)


# speedup vs baseline: 1.5099x; 1.0006x over previous
import jax
import jax.numpy as jnp
from jax import lax
from jax.experimental import pallas as pl
from jax.experimental.pallas import tpu as pltpu
from jax.experimental.pallas import tpu_sc as plsc

NC = 2
NS = 16

N_USERS = 16384
N_GROUPS = 1024
EMB = 64
HEMB = EMB // NC
N_BEH = 819200

B = 128
BEH_PER_W = N_BEH // NS
NBATCH = BEH_PER_W // B
SB = 20
NSB = NBATCH // SB
UPT = N_USERS // NS
UB = 128
NUB = UPT // UB
GPT = N_GROUPS // NS
KV = HEMB // 16


def _lane_bcast(v, l):
  idx = jnp.full((16, 1), l, dtype=jnp.int32)
  dn = lax.GatherDimensionNumbers(
      offset_dims=(), collapsed_slice_dims=(0,), start_index_map=(0,))
  return lax.gather(v, idx, dn, slice_sizes=(1,),
                    mode=lax.GatherScatterMode.PROMISE_IN_BOUNDS)


def _p1_body(bidx_h, bcnt_h, bseg_h, it_h, ub_out,
             bidx_s, bcnt_s, bseg_s, rows, urows, ubeh, gsem):
  cid = lax.axis_index("c")
  sid = lax.axis_index("s")

  zero16 = jnp.zeros((16,), jnp.float32)

  def zrow(r, _):
    for k in range(KV):
      urows[r, pl.ds(k * 16, 16)] = zero16
    return 0
  lax.fori_loop(0, GPT, zrow, 0)

  def zcopy(j, _):
    pltpu.sync_copy(urows.at[pl.ds(0, GPT)],
                    ubeh.at[pl.ds(sid * UPT + j * GPT, GPT)])
    return 0
  lax.fori_loop(0, UPT // GPT, zcopy, 0)

  plsc.subcore_barrier()

  def superstep(s, _):
    pltpu.sync_copy(bidx_h.at[sid, pl.ds(s * SB, SB)], bidx_s)
    pltpu.sync_copy(bcnt_h.at[sid, pl.ds(s * SB, SB)], bcnt_s)
    pltpu.sync_copy(bseg_h.at[sid, pl.ds(s * SB, SB)], bseg_s)

    def btrans(i, _):
      r = i // (B // 16)
      c0 = pl.multiple_of((i % (B // 16)) * 16, 16)
      iv = bidx_s[r, pl.ds(c0, 16)]
      cv = bcnt_s[r, pl.ds(c0, 16)]
      bcnt_s[r, pl.ds(c0, 16)] = jnp.where(iv != 0, cv, 0.0)
      bidx_s[r, pl.ds(c0, 16)] = iv * NC + cid
      return 0
    lax.fori_loop(0, SB * B // 16, btrans, 0)

    pltpu.async_copy(it_h.at[bidx_s.at[0]], rows.at[0], gsem)

    def pairstep(jj, _):
      for p in range(2):
        j = jj * 2 + p
        pltpu.make_async_copy(it_h.at[bidx_s.at[j]], rows.at[p], gsem).wait()

        @pl.when(j + 1 < SB)
        def _prefetch():
          pltpu.async_copy(it_h.at[bidx_s.at[j + 1]], rows.at[1 - p], gsem)

        def blk16(i, _, p=p, j=j):
          base = pl.multiple_of(i * 16, 16)
          cvec = bcnt_s[j, pl.ds(base, 16)]
          for l in range(16):
            cl = _lane_bcast(cvec, l)
            r = i * 16 + l
            for k in range(KV):
              rows[p, r, pl.ds(k * 16, 16)] = rows[p, r, pl.ds(k * 16, 16)] * cl
          return 0
        lax.fori_loop(0, B // 16, blk16, 0)
        pltpu.sync_copy(rows.at[p], ubeh.at[bseg_s.at[j]], add=True)
      return 0
    lax.fori_loop(0, SB // 2, pairstep, 0)
    return 0
  lax.fori_loop(0, NSB, superstep, 0)

  plsc.subcore_barrier()

  def udump(j, _):
    pltpu.sync_copy(ubeh.at[pl.ds(sid * UPT + j * UB, UB)],
                    urows.at[pl.ds(0, UB)])
    pltpu.sync_copy(urows.at[pl.ds(0, UB)],
                    ub_out.at[cid, pl.ds(sid * UPT + j * UB, UB)])
    return 0
  lax.fori_loop(0, UPT // UB, udump, 0)


def _p2_body(uidx_h, gidx_h, ut_h, ubeh_h, out_h,
             uidx, gidx, umsk, ubuf, urows, gacc, gsem):
  cid = lax.axis_index("c")
  sid = lax.axis_index("s")

  zero16 = jnp.zeros((16,), jnp.float32)

  def zrow(r, _):
    for k in range(KV):
      urows[r, pl.ds(k * 16, 16)] = zero16
    return 0
  lax.fori_loop(0, GPT, zrow, 0)
  pltpu.sync_copy(urows.at[pl.ds(0, GPT)], gacc.at[pl.ds(sid * GPT, GPT)])

  plsc.subcore_barrier()

  pltpu.sync_copy(uidx_h.at[sid], uidx)
  pltpu.sync_copy(gidx_h.at[sid], gidx)

  def utrans(i, _):
    r = i // (UB // 16)
    c0 = pl.multiple_of((i % (UB // 16)) * 16, 16)
    uv = uidx[r, pl.ds(c0, 16)]
    umsk[r, pl.ds(c0, 16)] = jnp.where(
        uv != 0, jnp.ones((16,), jnp.float32), jnp.zeros((16,), jnp.float32))
    uidx[r, pl.ds(c0, 16)] = uv * NC + cid
    return 0
  lax.fori_loop(0, UPT // 16, utrans, 0)

  def ubatch(j, _):
    pltpu.async_copy(ut_h.at[uidx.at[j]], urows, gsem).wait()
    pltpu.sync_copy(ubeh_h.at[cid, pl.ds(sid * UPT + j * UB, UB)], ubuf)

    def blk16(i, _):
      base = pl.multiple_of(i * 16, 16)
      mvec = umsk[j, pl.ds(base, 16)]
      for l in range(16):
        ml = _lane_bcast(mvec, l)
        r = i * 16 + l
        for k in range(KV):
          urows[r, pl.ds(k * 16, 16)] = (
              urows[r, pl.ds(k * 16, 16)] * ubuf[r, pl.ds(k * 16, 16)] * ml)
      return 0
    lax.fori_loop(0, UB // 16, blk16, 0)
    pltpu.sync_copy(urows, gacc.at[gidx.at[j]], add=True)
    return 0
  lax.fori_loop(0, NUB, ubatch, 0)

  plsc.subcore_barrier()

  pltpu.sync_copy(gacc.at[pl.ds(sid * GPT, GPT)], urows.at[pl.ds(0, GPT)])
  pltpu.sync_copy(urows.at[pl.ds(0, GPT)],
                  out_h.at[pl.ds(sid * GPT, GPT), pl.ds(cid * HEMB, HEMB)])


def kernel(user_ids, group_segment_ids, item_ids, counts, user_segment_ids,
           user_table, item_table):
  bidx = item_ids.astype(jnp.int32).reshape(NS, NBATCH, B)
  bcnt = counts.reshape(NS, NBATCH, B)
  bseg = user_segment_ids.astype(jnp.int32).reshape(NS, NBATCH, B)
  uidx = user_ids.astype(jnp.int32).reshape(NS, NUB, UB)
  gidx = group_segment_ids.astype(jnp.int32).reshape(NS, NUB, UB)
  ut2 = user_table.reshape(-1, HEMB)
  it2 = item_table.reshape(-1, HEMB)

  mesh = plsc.VectorSubcoreMesh(
      core_axis_name="c", subcore_axis_name="s", num_cores=NC, num_subcores=NS)
  p1 = pl.kernel(
      _p1_body,
      out_type=jax.ShapeDtypeStruct((NC, N_USERS, HEMB), jnp.float32),
      mesh=mesh,
      scratch_types=[
          pltpu.VMEM((SB, B), jnp.int32),
          pltpu.VMEM((SB, B), jnp.float32),
          pltpu.VMEM((SB, B), jnp.int32),
          pltpu.VMEM((2, B, HEMB), jnp.float32),
          pltpu.VMEM((UB, HEMB), jnp.float32),
          pltpu.VMEM_SHARED((N_USERS, HEMB), jnp.float32),
          pltpu.SemaphoreType.DMA,
      ],
      compiler_params=pltpu.CompilerParams(use_tc_tiling_on_sc=False),
  )
  ubeh_hbm = p1(bidx, bcnt, bseg, it2)

  p2 = pl.kernel(
      _p2_body,
      out_type=jax.ShapeDtypeStruct((N_GROUPS, EMB), jnp.float32),
      mesh=mesh,
      scratch_types=[
          pltpu.VMEM((NUB, UB), jnp.int32),
          pltpu.VMEM((NUB, UB), jnp.int32),
          pltpu.VMEM((NUB, UB), jnp.float32),
          pltpu.VMEM((UB, HEMB), jnp.float32),
          pltpu.VMEM((UB, HEMB), jnp.float32),
          pltpu.VMEM_SHARED((N_GROUPS, HEMB), jnp.float32),
          pltpu.SemaphoreType.DMA,
      ],
      compiler_params=pltpu.CompilerParams(use_tc_tiling_on_sc=False),
  )
  return p2(uidx, gidx, ut2, ubeh_hbm)
